# speedup vs baseline: 13.9947x; 13.9947x over previous
import functools

import jax
import jax.numpy as jnp
from jax import lax
from jax.experimental import pallas as pl
from jax.experimental.pallas import tpu as pltpu

N = 100000
B = 256
D = 128
W = 64
BN = 4000
NBLK = N // BN
NEG = -1e30


def _leaky(x):
    return jnp.where(x >= 0, x, 0.1 * x)


def _scores_body(wf_ref, ids_ref, w1_ref, w2_ref, tmp_ref, segmax_ref):
    i = pl.program_id(0)
    out = lax.dot_general(wf_ref[...], w1_ref[...], (((1,), (1,)), ((), ())),
                          preferred_element_type=jnp.float32)
    t = lax.dot_general(w2_ref[...], out, (((1,), (1,)), ((), ())),
                        preferred_element_type=jnp.float32)
    t = _leaky(t)
    tmp_ref[...] = t.reshape(1, 1, BN)
    ids = ids_ref[0, 0, :].reshape(1, BN)
    oh = lax.broadcasted_iota(jnp.int32, (B, BN), 0) == ids
    bmax = jnp.max(jnp.where(oh, t, NEG), axis=1).reshape(1, B)
    prev = jnp.where(i == 0, jnp.full((1, B), NEG, jnp.float32), segmax_ref[...])
    segmax_ref[...] = jnp.maximum(prev, bmax)


def _readout_body(feats_ref, ids_ref, tmp_ref, segmax_ref, out_ref,
                  acc_ref, den_ref, cnt_ref):
    i = pl.program_id(0)
    ids = ids_ref[0, 0, :].reshape(1, BN)
    oh = lax.broadcasted_iota(jnp.int32, (B, BN), 0) == ids
    oh_f = oh.astype(jnp.float32)
    t = tmp_ref[0, 0, :].reshape(1, BN)
    gmax = lax.dot_general(segmax_ref[...], oh_f, (((1,), (0,)), ((), ())),
                           preferred_element_type=jnp.float32)
    e = jnp.exp(t - gmax)
    w = oh_f * e
    bacc = lax.dot_general(w, feats_ref[...], (((1,), (0,)), ((), ())),
                           preferred_element_type=jnp.float32)
    bden = jnp.sum(w, axis=1, keepdims=True)
    bcnt = jnp.sum(oh_f, axis=1, keepdims=True)
    first = i == 0
    acc_ref[...] = jnp.where(first, 0.0, acc_ref[...]) + bacc
    den_ref[...] = jnp.where(first, 0.0, den_ref[...]) + bden
    cnt_ref[...] = jnp.where(first, 0.0, cnt_ref[...]) + bcnt

    @pl.when(i == NBLK - 1)
    def _():
        cnt = cnt_ref[...]
        mean_nodes = jnp.sum(cnt) / B
        scale = mean_nodes / (jnp.maximum(den_ref[...], 1e-30)
                              * jnp.maximum(cnt, 1.0))
        out_ref[...] = _leaky(acc_ref[...] * scale)


@functools.partial(jax.jit, static_argnames=("interpret",))
def kernel(node_feats, weights_feats, segment_ids, W1, W2, interpret=False):
    ids3 = segment_ids.astype(jnp.int32).reshape(NBLK, 1, BN)

    tmp, segmax = pl.pallas_call(
        _scores_body,
        grid=(NBLK,),
        in_specs=[
            pl.BlockSpec((BN, W), lambda i: (i, 0)),
            pl.BlockSpec((1, 1, BN), lambda i: (i, 0, 0)),
            pl.BlockSpec((2 * W, W), lambda i: (0, 0)),
            pl.BlockSpec((1, 2 * W), lambda i: (0, 0)),
        ],
        out_specs=[
            pl.BlockSpec((1, 1, BN), lambda i: (i, 0, 0)),
            pl.BlockSpec((1, B), lambda i: (0, 0)),
        ],
        out_shape=[
            jax.ShapeDtypeStruct((NBLK, 1, BN), jnp.float32),
            jax.ShapeDtypeStruct((1, B), jnp.float32),
        ],
        interpret=interpret,
    )(weights_feats, ids3, W1, W2)

    out = pl.pallas_call(
        _readout_body,
        grid=(NBLK,),
        in_specs=[
            pl.BlockSpec((BN, D), lambda i: (i, 0)),
            pl.BlockSpec((1, 1, BN), lambda i: (i, 0, 0)),
            pl.BlockSpec((1, 1, BN), lambda i: (i, 0, 0)),
            pl.BlockSpec((1, B), lambda i: (0, 0)),
        ],
        out_specs=pl.BlockSpec((B, D), lambda i: (0, 0)),
        out_shape=jax.ShapeDtypeStruct((B, D), jnp.float32),
        scratch_shapes=[
            pltpu.VMEM((B, D), jnp.float32),
            pltpu.VMEM((B, 1), jnp.float32),
            pltpu.VMEM((B, 1), jnp.float32),
        ],
        interpret=interpret,
    )(node_feats, ids3, tmp, segmax)

    return out


# speedup vs baseline: 15.3249x; 1.0951x over previous
import functools

import jax
import jax.numpy as jnp
from jax import lax
from jax.experimental import pallas as pl
from jax.experimental.pallas import tpu as pltpu

N = 100000
B = 256
D = 128
W = 64
BN = 4000
NBLK = N // BN
NEG = -1e30


def _leaky(x):
    return jnp.where(x >= 0, x, 0.1 * x)


def _body(wf_ref, ids_ref, feats_ref, w1_ref, w2_ref, out_ref,
          tmp_ref, tmax_ref, acc_ref, den_ref, cnt_ref):
    i = pl.program_id(0)

    @pl.when(i < NBLK)
    def _scores():
        out = lax.dot_general(wf_ref[...], w1_ref[...], (((1,), (1,)), ((), ())),
                              preferred_element_type=jnp.float32)
        t = lax.dot_general(w2_ref[...], out, (((1,), (1,)), ((), ())),
                            preferred_element_type=jnp.float32)
        t = _leaky(t)
        tmp_ref[pl.ds(i, 1)] = t.reshape(1, 1, BN)
        tmax_ref[0] = jnp.maximum(jnp.where(i == 0, NEG, tmax_ref[0]), jnp.max(t))
        ids = ids_ref[0, 0, :].reshape(1, BN)
        oh = lax.broadcasted_iota(jnp.int32, (B, BN), 0) == ids
        bcnt = jnp.sum(oh.astype(jnp.float32), axis=1, keepdims=True)
        cnt_ref[...] = jnp.where(i == 0, 0.0, cnt_ref[...]) + bcnt

    @pl.when(i >= NBLK)
    def _readout():
        j = i - NBLK
        ids = ids_ref[0, 0, :].reshape(1, BN)
        oh = lax.broadcasted_iota(jnp.int32, (B, BN), 0) == ids
        t = tmp_ref[pl.ds(j, 1)].reshape(1, BN)
        e = jnp.exp(t - tmax_ref[0])
        w = jnp.where(oh, e, 0.0)
        bacc = lax.dot_general(w.astype(jnp.bfloat16),
                               feats_ref[...].astype(jnp.bfloat16),
                               (((1,), (0,)), ((), ())),
                               preferred_element_type=jnp.float32)
        bden = jnp.sum(w, axis=1, keepdims=True)
        first = j == 0
        acc_ref[...] = jnp.where(first, 0.0, acc_ref[...]) + bacc
        den_ref[...] = jnp.where(first, 0.0, den_ref[...]) + bden

        @pl.when(j == NBLK - 1)
        def _():
            mean_nodes = float(N) / float(B)
            scale = mean_nodes / (jnp.maximum(den_ref[...], 1e-30)
                                  * jnp.maximum(cnt_ref[...], 1.0))
            out_ref[...] = _leaky(acc_ref[...] * scale)


@functools.partial(jax.jit, static_argnames=("interpret",))
def kernel(node_feats, weights_feats, segment_ids, W1, W2, interpret=False):
    ids3 = segment_ids.astype(jnp.int32).reshape(NBLK, 1, BN)

    out = pl.pallas_call(
        _body,
        grid=(2 * NBLK,),
        in_specs=[
            pl.BlockSpec((BN, W), lambda i: (jnp.minimum(i, NBLK - 1), 0)),
            pl.BlockSpec((1, 1, BN),
                         lambda i: (jnp.where(i < NBLK, i, i - NBLK), 0, 0)),
            pl.BlockSpec((BN, D), lambda i: (jnp.maximum(i - NBLK, 0), 0)),
            pl.BlockSpec((2 * W, W), lambda i: (0, 0)),
            pl.BlockSpec((1, 2 * W), lambda i: (0, 0)),
        ],
        out_specs=pl.BlockSpec((B, D), lambda i: (0, 0)),
        out_shape=jax.ShapeDtypeStruct((B, D), jnp.float32),
        scratch_shapes=[
            pltpu.VMEM((NBLK, 1, BN), jnp.float32),
            pltpu.SMEM((1,), jnp.float32),
            pltpu.VMEM((B, D), jnp.float32),
            pltpu.VMEM((B, 1), jnp.float32),
            pltpu.VMEM((B, 1), jnp.float32),
        ],
        interpret=interpret,
    )(weights_feats, ids3, node_feats, W1, W2)

    return out
